# initial kernel scaffold (unmeasured)
import jax
import jax.numpy as jnp
from jax import lax
from jax.experimental import pallas as pl
from jax.experimental.pallas import tpu as pltpu

N_DEV = 16


def kernel(x, w_mat, scale_x, scale_w):
    m_per, k = x.shape
    n = w_mat.shape[1]
    n_per = n // N_DEV
    m_tot = m_per * N_DEV

    sx = scale_x.reshape(1, 1)
    sw = scale_w.reshape(1, 1)

    def body(x_ref, w_ref, sx_ref, sw_ref, out_ref,
             x_bf, w_buf, chunks, copy_sems, send_sems, recv_sems):
        my_i = lax.axis_index("i")
        s = sx_ref[0, 0] * sw_ref[0, 0]
        x_bf[...] = x_ref[...].astype(jnp.bfloat16)

        def w_copy(d, slot):
            j = lax.rem(my_i + d, N_DEV)
            return pltpu.make_async_copy(
                w_ref.at[:, pl.ds(j * n_per, n_per)],
                w_buf.at[slot],
                copy_sems.at[slot],
            )

        def send_desc(d):
            j = lax.rem(my_i + d, N_DEV)
            return pltpu.make_async_remote_copy(
                src_ref=chunks.at[d],
                dst_ref=out_ref.at[pl.ds(my_i * m_per, m_per), :],
                send_sem=send_sems.at[d],
                recv_sem=recv_sems.at[d],
                device_id=(j,),
                device_id_type=pl.DeviceIdType.MESH,
            )

        def recv_desc(d):
            src = lax.rem(my_i + (N_DEV - d), N_DEV)
            return pltpu.make_async_remote_copy(
                src_ref=chunks.at[d],
                dst_ref=out_ref.at[pl.ds(src * m_per, m_per), :],
                send_sem=send_sems.at[d],
                recv_sem=recv_sems.at[d],
                device_id=(src,),
                device_id_type=pl.DeviceIdType.MESH,
            )

        w_copy(0, 0).start()
        for d in range(N_DEV):
            slot = d % 2
            w_copy(d, slot).wait()
            if d + 1 < N_DEV:
                w_copy(d + 1, (d + 1) % 2).start()
            acc = jnp.dot(x_bf[...], w_buf[slot].astype(jnp.bfloat16),
                          preferred_element_type=jnp.float32)
            yv = jnp.maximum(acc * s, 0.0)
            if d == 0:
                out_ref[pl.ds(my_i * m_per, m_per), :] = yv
            else:
                chunks[d, :, :] = yv
                send_desc(d).start()

        for d in range(1, N_DEV):
            recv_desc(d).wait_recv()
        for d in range(1, N_DEV):
            send_desc(d).wait_send()

    return pl.pallas_call(
        body,
        out_shape=jax.ShapeDtypeStruct((m_tot, n_per), jnp.float32),
        in_specs=[
            pl.BlockSpec(memory_space=pltpu.VMEM),
            pl.BlockSpec(memory_space=pltpu.ANY),
            pl.BlockSpec(memory_space=pltpu.SMEM),
            pl.BlockSpec(memory_space=pltpu.SMEM),
        ],
        out_specs=pl.BlockSpec(memory_space=pltpu.VMEM),
        scratch_shapes=[
            pltpu.VMEM((m_per, k), jnp.bfloat16),
            pltpu.VMEM((2, k, n_per), jnp.float32),
            pltpu.VMEM((N_DEV, m_per, n_per), jnp.float32),
            pltpu.SemaphoreType.DMA((2,)),
            pltpu.SemaphoreType.DMA((N_DEV,)),
            pltpu.SemaphoreType.DMA((N_DEV,)),
        ],
        compiler_params=pltpu.CompilerParams(collective_id=0),
    )(x, w_mat, sx, sw)


# baseline (device time: 109397 ns/iter reference)
import jax
import jax.numpy as jnp
from jax import lax
from jax.experimental import pallas as pl
from jax.experimental.pallas import tpu as pltpu

N_DEV = 16


def kernel(x, w_mat, scale_x, scale_w):
    m_per, k = x.shape
    n = w_mat.shape[1]
    n_per = n // N_DEV
    m_tot = m_per * N_DEV

    sx = scale_x.reshape(1, 1)
    sw = scale_w.reshape(1, 1)

    def body(x_ref, w_ref, sx_ref, sw_ref, out_ref,
             x_bf, w_buf, chunks, copy_sems, send_sems, recv_sems):
        my_i = lax.axis_index("i")
        s = sx_ref[0, 0] * sw_ref[0, 0]
        x_bf[...] = x_ref[...].astype(jnp.bfloat16)

        def w_copy(d, slot):
            j = lax.rem(my_i + d, N_DEV)
            return pltpu.make_async_copy(
                w_ref.at[:, pl.ds(j * n_per, n_per)],
                w_buf.at[slot],
                copy_sems.at[slot],
            )

        def send_desc(d):
            j = lax.rem(my_i + d, N_DEV)
            return pltpu.make_async_remote_copy(
                src_ref=chunks.at[d],
                dst_ref=out_ref.at[pl.ds(my_i * m_per, m_per), :],
                send_sem=send_sems.at[d],
                recv_sem=recv_sems.at[d],
                device_id=(j,),
                device_id_type=pl.DeviceIdType.MESH,
            )

        def recv_desc(d):
            src = lax.rem(my_i + (N_DEV - d), N_DEV)
            return pltpu.make_async_remote_copy(
                src_ref=chunks.at[d],
                dst_ref=out_ref.at[pl.ds(src * m_per, m_per), :],
                send_sem=send_sems.at[d],
                recv_sem=recv_sems.at[d],
                device_id=(src,),
                device_id_type=pl.DeviceIdType.MESH,
            )

        w_copy(0, 0).start()
        for d in range(N_DEV):
            slot = d % 2
            w_copy(d, slot).wait()
            if d + 1 < N_DEV:
                w_copy(d + 1, (d + 1) % 2).start()
            acc = jnp.dot(x_bf[...], w_buf[slot].astype(jnp.bfloat16),
                          preferred_element_type=jnp.float32)
            yv = jnp.maximum(acc * s, 0.0)
            if d == 0:
                out_ref[pl.ds(my_i * m_per, m_per), :] = yv
            else:
                chunks[d, :, :] = yv
                send_desc(d).start()

        for d in range(1, N_DEV):
            recv_desc(d).wait_recv()
        for d in range(1, N_DEV):
            send_desc(d).wait_send()

    return pl.pallas_call(
        body,
        out_shape=jax.ShapeDtypeStruct((m_tot, n_per), jnp.float32),
        in_specs=[
            pl.BlockSpec(memory_space=pltpu.VMEM),
            pl.BlockSpec(memory_space=pl.ANY),
            pl.BlockSpec(memory_space=pltpu.SMEM),
            pl.BlockSpec(memory_space=pltpu.SMEM),
        ],
        out_specs=pl.BlockSpec(memory_space=pltpu.VMEM),
        scratch_shapes=[
            pltpu.VMEM((m_per, k), jnp.bfloat16),
            pltpu.VMEM((2, k, n_per), jnp.float32),
            pltpu.VMEM((N_DEV, m_per, n_per), jnp.float32),
            pltpu.SemaphoreType.DMA((2,)),
            pltpu.SemaphoreType.DMA((N_DEV,)),
            pltpu.SemaphoreType.DMA((N_DEV,)),
        ],
    )(x, w_mat, sx, sw)


# device time: 80001 ns/iter; 1.3674x vs baseline; 1.3674x over previous
import jax
import jax.numpy as jnp
from jax import lax
from jax.experimental import pallas as pl
from jax.experimental.pallas import tpu as pltpu

N_DEV = 16


def kernel(x, w_mat, scale_x, scale_w):
    m_per, k = x.shape
    n = w_mat.shape[1]
    n_per = n // N_DEV
    m_tot = m_per * N_DEV

    sx = scale_x.reshape(1, 1)
    sw = scale_w.reshape(1, 1)

    def body(x_ref, w_ref, sx_ref, sw_ref, out_ref,
             x_bf, w_buf, chunks, recv_buf, copy_sems, send_sems, recv_sems):
        my_i = lax.axis_index("i")
        s = sx_ref[0, 0] * sw_ref[0, 0]
        x_bf[...] = x_ref[...].astype(jnp.bfloat16)

        def w_copy(d, slot):
            j = lax.rem(my_i + d, N_DEV)
            return pltpu.make_async_copy(
                w_ref.at[:, pl.ds(j * n_per, n_per)],
                w_buf.at[slot],
                copy_sems.at[slot],
            )

        def send_desc(d):
            j = lax.rem(my_i + d, N_DEV)
            return pltpu.make_async_remote_copy(
                src_ref=chunks.at[d],
                dst_ref=recv_buf.at[d],
                send_sem=send_sems.at[d],
                recv_sem=recv_sems.at[d],
                device_id=(j,),
                device_id_type=pl.DeviceIdType.MESH,
            )

        w_copy(0, 0).start()
        for d in range(N_DEV):
            slot = d % 2
            w_copy(d, slot).wait()
            if d + 1 < N_DEV:
                w_copy(d + 1, (d + 1) % 2).start()
            acc = jnp.dot(x_bf[...], w_buf[slot].astype(jnp.bfloat16),
                          preferred_element_type=jnp.float32)
            yv = jnp.maximum(acc * s, 0.0)
            if d == 0:
                out_ref[pl.ds(my_i * m_per, m_per), :] = yv
            else:
                chunks[d, :, :] = yv.astype(jnp.bfloat16)
                send_desc(d).start()

        for d in range(1, N_DEV):
            src = lax.rem(my_i + (N_DEV - d), N_DEV)
            send_desc(d).wait_recv()
            out_ref[pl.ds(src * m_per, m_per), :] = (
                recv_buf[d, :, :].astype(jnp.float32))
        for d in range(1, N_DEV):
            send_desc(d).wait_send()

    return pl.pallas_call(
        body,
        out_shape=jax.ShapeDtypeStruct((m_tot, n_per), jnp.float32),
        in_specs=[
            pl.BlockSpec(memory_space=pltpu.VMEM),
            pl.BlockSpec(memory_space=pl.ANY),
            pl.BlockSpec(memory_space=pltpu.SMEM),
            pl.BlockSpec(memory_space=pltpu.SMEM),
        ],
        out_specs=pl.BlockSpec(memory_space=pltpu.VMEM),
        scratch_shapes=[
            pltpu.VMEM((m_per, k), jnp.bfloat16),
            pltpu.VMEM((2, k, n_per), jnp.float32),
            pltpu.VMEM((N_DEV, m_per, n_per), jnp.bfloat16),
            pltpu.VMEM((N_DEV, m_per, n_per), jnp.bfloat16),
            pltpu.SemaphoreType.DMA((2,)),
            pltpu.SemaphoreType.DMA((N_DEV,)),
            pltpu.SemaphoreType.DMA((N_DEV,)),
        ],
    )(x, w_mat, sx, sw)
